# baseline (speedup 1.0000x reference)
import functools

import jax
import jax.numpy as jnp
from jax.experimental import pallas as pl
from jax.experimental.pallas import tpu as pltpu

_C = 300000
_K = 90000
_D = 128
_BLK = 2000


def _sim_body(e1_ref, e2_ref, out_ref):
    a = e1_ref[...]
    b = e2_ref[...]
    num = jnp.sum(a * b, axis=1)
    na = jnp.sqrt(jnp.sum(a * a, axis=1))
    nb = jnp.sqrt(jnp.sum(b * b, axis=1))
    out_ref[0, 0, :] = num / jnp.maximum(na * nb, 1e-8)


def _sims(e1, e2):
    grid = _C // _BLK
    out = pl.pallas_call(
        _sim_body,
        grid=(grid,),
        in_specs=[
            pl.BlockSpec((_BLK, _D), lambda i: (i, 0)),
            pl.BlockSpec((_BLK, _D), lambda i: (i, 0)),
        ],
        out_specs=pl.BlockSpec((1, 1, _BLK), lambda i: (i, 0, 0)),
        out_shape=jax.ShapeDtypeStruct((grid, 1, _BLK), jnp.float32),
    )(e1, e2)
    return out.reshape(-1)


def kernel(embeddings, positive_pairs, candidate_negatives):
    del positive_pairs
    e1 = jnp.take(embeddings, candidate_negatives[:, 0], axis=0)
    e2 = jnp.take(embeddings, candidate_negatives[:, 1], axis=0)
    sims = _sims(e1, e2)
    _, idx = jax.lax.top_k(sims, _K)
    return jnp.take(candidate_negatives, idx, axis=0)


# speedup vs baseline: 1.1993x; 1.1993x over previous
import functools

import jax
import jax.numpy as jnp
from jax import lax
from jax.experimental import pallas as pl
from jax.experimental.pallas import tpu as pltpu
from jax.experimental.pallas import tpu_sc as plsc

_N = 100000
_D = 128
_C = 300000
_K = 90000
_CP = 307200
_N2 = 524288
_ROWS = _N2 // 128
_CHR = 32
_SBLK = 2048
_GP = 90112


def _sc_gather(table, idx, chunk):
    B = idx.shape[0]
    NC, NS = 2, 16
    NW = NC * NS
    bpw = B // NW
    nch = bpw // chunk
    dt = table.dtype
    two_d = table.ndim == 2
    row_shape = (chunk, table.shape[1]) if two_d else (chunk,)
    out_shape = (B, table.shape[1]) if two_d else (B,)
    mesh = plsc.VectorSubcoreMesh(core_axis_name="c", subcore_axis_name="s")

    @functools.partial(
        pl.kernel,
        mesh=mesh,
        out_type=jax.ShapeDtypeStruct(out_shape, dt),
        scratch_types=[
            pltpu.VMEM((chunk,), jnp.int32),
            pltpu.VMEM(row_shape, dt),
            pltpu.SemaphoreType.DMA,
        ],
    )
    def k(table_hbm, idx_hbm, out_hbm, idx_v, rows_v, sem):
        wid = lax.axis_index("s") * NC + lax.axis_index("c")
        base = wid * bpw

        def step(ci, carry):
            off = base + ci * chunk
            pltpu.sync_copy(idx_hbm.at[pl.ds(off, chunk)], idx_v)
            pltpu.async_copy(table_hbm.at[idx_v], rows_v, sem).wait()
            pltpu.sync_copy(rows_v, out_hbm.at[pl.ds(off, chunk)])
            return carry

        lax.fori_loop(0, nch, step, 0)

    return k(table, idx)


def _simkey_body(a_ref, b_ref, out_ref):
    a, b = a_ref[...], b_ref[...]
    num = jnp.sum((a * b).T, axis=0)
    na = jnp.sqrt(jnp.sum((a * a).T, axis=0))
    nb = jnp.sqrt(jnp.sum((b * b).T, axis=0))
    sim = num / jnp.maximum(na * nb, 1e-8)
    s = lax.bitcast_convert_type(sim, jnp.int32)
    t = jnp.where(s < 0, s ^ jnp.int32(0x7FFFFFFF), s)
    key = (t ^ jnp.int32(-1)).reshape(1, 1, _SBLK)
    e = (pl.program_id(0) * _SBLK
         + lax.broadcasted_iota(jnp.int32, (1, 1, _SBLK), 2))
    out_ref[...] = jnp.where(e < _C, key, jnp.int32(0x7FFFFFFF))


def _simkey(e1, e2):
    grid = _CP // _SBLK
    out = pl.pallas_call(
        _simkey_body,
        grid=(grid,),
        in_specs=[
            pl.BlockSpec((_SBLK, _D), lambda i: (i, 0)),
            pl.BlockSpec((_SBLK, _D), lambda i: (i, 0)),
        ],
        out_specs=pl.BlockSpec((1, 1, _SBLK), lambda i: (i, 0, 0)),
        out_shape=jax.ShapeDtypeStruct((grid, 1, _SBLK), jnp.int32),
    )(e1, e2)
    return out.reshape(-1)


def _gt(ka, ia, kb, ib):
    return (ka > kb) | ((ka == kb) & (ia > ib))


def _substage(kv, iv, e, s, m):
    if s >= 128:
        sr = s // 128
        kd, ku = jnp.roll(kv, -sr, axis=0), jnp.roll(kv, sr, axis=0)
        idn, iup = jnp.roll(iv, -sr, axis=0), jnp.roll(iv, sr, axis=0)
    else:
        kd, ku = jnp.roll(kv, -s, axis=1), jnp.roll(kv, s, axis=1)
        idn, iup = jnp.roll(iv, -s, axis=1), jnp.roll(iv, s, axis=1)
    lo = (e & s) == 0
    ko = jnp.where(lo, kd, ku)
    io = jnp.where(lo, idn, iup)
    desc = (e & m) != 0
    take = ((lo & _gt(kv, iv, ko, io)) | (~lo & _gt(ko, io, kv, iv))) ^ desc
    return jnp.where(take, ko, kv), jnp.where(take, io, iv)


def _local_body(stages, ks_ref, is_ref, ko_ref, io_ref):
    base = pl.program_id(0) * (_CHR * 128)
    e = (base
         + 128 * lax.broadcasted_iota(jnp.int32, (_CHR, 128), 0)
         + lax.broadcasted_iota(jnp.int32, (_CHR, 128), 1))
    kv, iv = ks_ref[...], is_ref[...]
    for (s, m) in stages:
        kv, iv = _substage(kv, iv, e, s, m)
    ko_ref[...] = kv
    io_ref[...] = iv


def _cross_body(s, m, k_ref, i_ref, ko_ref, io_ref):
    desc = ((pl.program_id(0) * (2 * s)) & m) != 0
    half = s // 128
    klo, khi = k_ref[:half], k_ref[half:]
    ilo, ihi = i_ref[:half], i_ref[half:]
    swap = _gt(klo, ilo, khi, ihi) ^ desc
    ko_ref[:half] = jnp.where(swap, khi, klo)
    ko_ref[half:] = jnp.where(swap, klo, khi)
    io_ref[:half] = jnp.where(swap, ihi, ilo)
    io_ref[half:] = jnp.where(swap, ilo, ihi)


def _sort_call(body, grid, rows, K, I):
    bs = pl.BlockSpec((rows, 128), lambda i: (i, 0))
    return pl.pallas_call(
        body,
        grid=(grid,),
        in_specs=[bs, bs],
        out_specs=[bs, bs],
        out_shape=[jax.ShapeDtypeStruct((_ROWS, 128), jnp.int32)] * 2,
        input_output_aliases={0: 0, 1: 1},
    )(K, I)


def _sort(K, I):
    chunk_e = _CHR * 128
    local_stages = []
    m = 2
    while m <= chunk_e:
        s = m // 2
        while s >= 1:
            local_stages.append((s, m))
            s //= 2
        m *= 2
    K, I = _sort_call(functools.partial(_local_body, local_stages),
                      _ROWS // _CHR, _CHR, K, I)
    m = 2 * chunk_e
    while m <= _N2:
        s = m // 2
        while s >= chunk_e:
            K, I = _sort_call(functools.partial(_cross_body, s, m),
                              _N2 // (2 * s), 2 * s // 128, K, I)
            s //= 2
        merge_stages = []
        while s >= 1:
            merge_stages.append((s, m))
            s //= 2
        K, I = _sort_call(functools.partial(_local_body, merge_stages),
                          _ROWS // _CHR, _CHR, K, I)
        m *= 2
    return K, I


def kernel(embeddings, positive_pairs, candidate_negatives):
    del positive_pairs
    cn = candidate_negatives
    pad_idx = (jnp.arange(_CP - _C, dtype=jnp.int32) * 7919) % _N
    i1 = jnp.concatenate([cn[:, 0], pad_idx])
    i2 = jnp.concatenate([cn[:, 1], pad_idx])
    e1 = _sc_gather(embeddings, i1, 600)
    e2 = _sc_gather(embeddings, i2, 600)
    K0 = _simkey(e1, e2)
    K = jnp.concatenate(
        [K0, jnp.full((_N2 - _CP,), 0x7FFFFFFF, jnp.int32)]).reshape(_ROWS, 128)
    I = jnp.arange(_N2, dtype=jnp.int32).reshape(_ROWS, 128)
    _, Is = _sort(K, I)
    top = Is.reshape(-1)[:_K]
    topp = jnp.concatenate(
        [top, (jnp.arange(_GP - _K, dtype=jnp.int32) * 13) % _C])
    g0 = _sc_gather(cn[:, 0] + 0, topp, _GP // 32)
    g1 = _sc_gather(cn[:, 1] + 0, topp, _GP // 32)
    return jnp.stack([g0[:_K], g1[:_K]], axis=1)


# speedup vs baseline: 1.7711x; 1.4768x over previous
import functools

import jax
import jax.numpy as jnp
from jax import lax
from jax.experimental import pallas as pl
from jax.experimental.pallas import tpu as pltpu
from jax.experimental.pallas import tpu_sc as plsc

_N = 100000
_D = 128
_C = 300000
_K = 90000
_CP = 307200
_N2 = 524288
_ROWS = _N2 // 128
_CHR = 32
_SBLK = 2048
_GP = 90112


def _sc_gather(table, idx, chunk):
    B = idx.shape[0]
    NC, NS = 2, 16
    NW = NC * NS
    bpw = B // NW
    nch = bpw // chunk
    dt = table.dtype
    two_d = table.ndim == 2
    row_shape = (chunk, table.shape[1]) if two_d else (chunk,)
    out_shape = (B, table.shape[1]) if two_d else (B,)
    mesh = plsc.VectorSubcoreMesh(core_axis_name="c", subcore_axis_name="s")

    @functools.partial(
        pl.kernel,
        mesh=mesh,
        out_type=jax.ShapeDtypeStruct(out_shape, dt),
        scratch_types=[
            pltpu.VMEM((chunk,), jnp.int32),
            pltpu.VMEM(row_shape, dt),
            pltpu.SemaphoreType.DMA,
        ],
    )
    def k(table_hbm, idx_hbm, out_hbm, idx_v, rows_v, sem):
        wid = lax.axis_index("s") * NC + lax.axis_index("c")
        base = wid * bpw

        def step(ci, carry):
            off = base + ci * chunk
            pltpu.sync_copy(idx_hbm.at[pl.ds(off, chunk)], idx_v)
            pltpu.async_copy(table_hbm.at[idx_v], rows_v, sem).wait()
            pltpu.sync_copy(rows_v, out_hbm.at[pl.ds(off, chunk)])
            return carry

        lax.fori_loop(0, nch, step, 0)

    return k(table, idx)


def _simkey_body(a_ref, b_ref, out_ref):
    a, b = a_ref[...], b_ref[...]
    num = jnp.sum((a * b).T, axis=0)
    na = jnp.sqrt(jnp.sum((a * a).T, axis=0))
    nb = jnp.sqrt(jnp.sum((b * b).T, axis=0))
    sim = num / jnp.maximum(na * nb, 1e-8)
    s = lax.bitcast_convert_type(sim, jnp.int32)
    t = jnp.where(s < 0, s ^ jnp.int32(0x7FFFFFFF), s)
    key = (t ^ jnp.int32(-1)).reshape(1, 1, _SBLK)
    e = (pl.program_id(0) * _SBLK
         + lax.broadcasted_iota(jnp.int32, (1, 1, _SBLK), 2))
    out_ref[...] = jnp.where(e < _C, key, jnp.int32(0x7FFFFFFF))


def _simkey(e1, e2):
    grid = _CP // _SBLK
    out = pl.pallas_call(
        _simkey_body,
        grid=(grid,),
        in_specs=[
            pl.BlockSpec((_SBLK, _D), lambda i: (i, 0)),
            pl.BlockSpec((_SBLK, _D), lambda i: (i, 0)),
        ],
        out_specs=pl.BlockSpec((1, 1, _SBLK), lambda i: (i, 0, 0)),
        out_shape=jax.ShapeDtypeStruct((grid, 1, _SBLK), jnp.int32),
    )(e1, e2)
    return out.reshape(-1)


def _gt(ka, ia, kb, ib):
    return (ka > kb) | ((ka == kb) & (ia > ib))


def _substage(kv, iv, e, s, m):
    if s >= 128:
        sr = s // 128
        kd, ku = jnp.roll(kv, -sr, axis=0), jnp.roll(kv, sr, axis=0)
        idn, iup = jnp.roll(iv, -sr, axis=0), jnp.roll(iv, sr, axis=0)
    else:
        kd, ku = jnp.roll(kv, -s, axis=1), jnp.roll(kv, s, axis=1)
        idn, iup = jnp.roll(iv, -s, axis=1), jnp.roll(iv, s, axis=1)
    lo = (e & s) == 0
    ko = jnp.where(lo, kd, ku)
    io = jnp.where(lo, idn, iup)
    desc = (e & m) != 0
    take = ((lo & _gt(kv, iv, ko, io)) | (~lo & _gt(ko, io, kv, iv))) ^ desc
    return jnp.where(take, ko, kv), jnp.where(take, io, iv)


def _local_body(stages, rows, ks_ref, is_ref, ko_ref, io_ref):
    base = pl.program_id(0) * (rows * 128)
    e = (base
         + 128 * lax.broadcasted_iota(jnp.int32, (rows, 128), 0)
         + lax.broadcasted_iota(jnp.int32, (rows, 128), 1))
    kv, iv = ks_ref[...], is_ref[...]
    for (s, m) in stages:
        kv, iv = _substage(kv, iv, e, s, m)
    ko_ref[...] = kv
    io_ref[...] = iv


def _cross_body(s, m, k_ref, i_ref, ko_ref, io_ref):
    desc = ((pl.program_id(0) * (2 * s)) & m) != 0
    half = s // 128
    klo, khi = k_ref[:half], k_ref[half:]
    ilo, ihi = i_ref[:half], i_ref[half:]
    swap = _gt(klo, ilo, khi, ihi) ^ desc
    ko_ref[:half] = jnp.where(swap, khi, klo)
    ko_ref[half:] = jnp.where(swap, klo, khi)
    io_ref[:half] = jnp.where(swap, ihi, ilo)
    io_ref[half:] = jnp.where(swap, ilo, ihi)


def _sort_call(body, grid, rows, K, I):
    bs = pl.BlockSpec((rows, 128), lambda i: (i, 0))
    return pl.pallas_call(
        body,
        grid=(grid,),
        in_specs=[bs, bs],
        out_specs=[bs, bs],
        out_shape=[jax.ShapeDtypeStruct((_ROWS, 128), jnp.int32)] * 2,
        input_output_aliases={0: 0, 1: 1},
    )(K, I)


def _stages_down(m, s_hi):
    s, out = s_hi, []
    while s >= 1:
        out.append((s, m))
        s //= 2
    return out


def _fused_local(stages, rows, K, I):
    return _sort_call(functools.partial(_local_body, stages, rows),
                      _ROWS // rows, rows, K, I)


def _sort(K, I):
    stages = []
    m = 2
    while m <= 4096:
        stages += _stages_down(m, m // 2)
        m *= 2
    K, I = _fused_local(stages, _CHR, K, I)
    stages = []
    for m in (8192, 16384, 32768, 65536):
        stages += _stages_down(m, m // 2)
    K, I = _fused_local(stages, 512, K, I)
    K, I = _fused_local(_stages_down(131072, 65536), 1024, K, I)
    K, I = _sort_call(functools.partial(_cross_body, 131072, 262144),
                      2, 2048, K, I)
    K, I = _fused_local(_stages_down(262144, 65536), 1024, K, I)
    K, I = _sort_call(functools.partial(_cross_body, 262144, 524288),
                      1, 4096, K, I)
    K, I = _sort_call(functools.partial(_cross_body, 131072, 524288),
                      2, 2048, K, I)
    K, I = _fused_local(_stages_down(524288, 65536), 1024, K, I)
    return K, I


def kernel(embeddings, positive_pairs, candidate_negatives):
    del positive_pairs
    cn = candidate_negatives
    pad_idx = (jnp.arange(_CP - _C, dtype=jnp.int32) * 7919) % _N
    i1 = jnp.concatenate([cn[:, 0], pad_idx])
    i2 = jnp.concatenate([cn[:, 1], pad_idx])
    e1 = _sc_gather(embeddings, i1, 600)
    e2 = _sc_gather(embeddings, i2, 600)
    K0 = _simkey(e1, e2)
    K = jnp.concatenate(
        [K0, jnp.full((_N2 - _CP,), 0x7FFFFFFF, jnp.int32)]).reshape(_ROWS, 128)
    I = jnp.arange(_N2, dtype=jnp.int32).reshape(_ROWS, 128)
    _, Is = _sort(K, I)
    top = Is.reshape(-1)[:_K]
    topp = jnp.concatenate(
        [top, (jnp.arange(_GP - _K, dtype=jnp.int32) * 13) % _C])
    g0 = _sc_gather(cn[:, 0] + 0, topp, _GP // 32)
    g1 = _sc_gather(cn[:, 1] + 0, topp, _GP // 32)
    return jnp.stack([g0[:_K], g1[:_K]], axis=1)


# speedup vs baseline: 1.8182x; 1.0266x over previous
import functools

import jax
import jax.numpy as jnp
from jax import lax
from jax.experimental import pallas as pl
from jax.experimental.pallas import tpu as pltpu
from jax.experimental.pallas import tpu_sc as plsc

_N = 100000
_D = 128
_C = 300000
_K = 90000
_CP = 307200
_N2 = 524288
_ROWS = _N2 // 128
_CHR = 32
_SBLK = 2048
_GP = 90112


def _sc_gather(table, idx, chunk):
    B = idx.shape[0]
    NC, NS = 2, 16
    NW = NC * NS
    bpw = B // NW
    nch = bpw // chunk
    dt = table.dtype
    two_d = table.ndim == 2
    row_shape = (chunk, table.shape[1]) if two_d else (chunk,)
    out_shape = (B, table.shape[1]) if two_d else (B,)
    mesh = plsc.VectorSubcoreMesh(core_axis_name="c", subcore_axis_name="s")

    @functools.partial(
        pl.kernel,
        mesh=mesh,
        out_type=jax.ShapeDtypeStruct(out_shape, dt),
        scratch_types=[
            pltpu.VMEM((chunk,), jnp.int32),
            pltpu.VMEM(row_shape, dt),
            pltpu.SemaphoreType.DMA,
        ],
    )
    def k(table_hbm, idx_hbm, out_hbm, idx_v, rows_v, sem):
        wid = lax.axis_index("s") * NC + lax.axis_index("c")
        base = wid * bpw

        def step(ci, carry):
            off = base + ci * chunk
            pltpu.sync_copy(idx_hbm.at[pl.ds(off, chunk)], idx_v)
            pltpu.async_copy(table_hbm.at[idx_v], rows_v, sem).wait()
            pltpu.sync_copy(rows_v, out_hbm.at[pl.ds(off, chunk)])
            return carry

        lax.fori_loop(0, nch, step, 0)

    return k(table, idx)


def _simkey_body(a_ref, b_ref, out_ref):
    a, b = a_ref[...], b_ref[...]
    num = jnp.sum((a * b).T, axis=0)
    na = jnp.sqrt(jnp.sum((a * a).T, axis=0))
    nb = jnp.sqrt(jnp.sum((b * b).T, axis=0))
    sim = num / jnp.maximum(na * nb, 1e-8)
    s = lax.bitcast_convert_type(sim, jnp.int32)
    t = jnp.where(s < 0, s ^ jnp.int32(0x7FFFFFFF), s)
    key = (t ^ jnp.int32(-1)).reshape(1, 1, _SBLK)
    e = (pl.program_id(0) * _SBLK
         + lax.broadcasted_iota(jnp.int32, (1, 1, _SBLK), 2))
    out_ref[...] = jnp.where(e < _C, key, jnp.int32(0x7FFFFFFF))


def _simkey(e1, e2):
    grid = _CP // _SBLK
    out = pl.pallas_call(
        _simkey_body,
        grid=(grid,),
        in_specs=[
            pl.BlockSpec((_SBLK, _D), lambda i: (i, 0)),
            pl.BlockSpec((_SBLK, _D), lambda i: (i, 0)),
        ],
        out_specs=pl.BlockSpec((1, 1, _SBLK), lambda i: (i, 0, 0)),
        out_shape=jax.ShapeDtypeStruct((grid, 1, _SBLK), jnp.int32),
    )(e1, e2)
    return out.reshape(-1)


def _gt(ka, ia, kb, ib):
    return (ka > kb) | ((ka == kb) & (ia > ib))


def _substage(kv, iv, e, s, m):
    if s >= 128:
        sr = s // 128
        kd, ku = jnp.roll(kv, -sr, axis=0), jnp.roll(kv, sr, axis=0)
        idn, iup = jnp.roll(iv, -sr, axis=0), jnp.roll(iv, sr, axis=0)
    else:
        kd, ku = jnp.roll(kv, -s, axis=1), jnp.roll(kv, s, axis=1)
        idn, iup = jnp.roll(iv, -s, axis=1), jnp.roll(iv, s, axis=1)
    lo = (e & s) == 0
    ko = jnp.where(lo, kd, ku)
    io = jnp.where(lo, idn, iup)
    desc = (e & m) != 0
    take = ((lo & _gt(kv, iv, ko, io)) | (~lo & _gt(ko, io, kv, iv))) ^ desc
    return jnp.where(take, ko, kv), jnp.where(take, io, iv)


def _local_body(stages, rows, ks_ref, is_ref, ko_ref, io_ref):
    base = pl.program_id(0) * (rows * 128)
    e = (base
         + 128 * lax.broadcasted_iota(jnp.int32, (rows, 128), 0)
         + lax.broadcasted_iota(jnp.int32, (rows, 128), 1))
    kv, iv = ks_ref[...], is_ref[...]
    for (s, m) in stages:
        kv, iv = _substage(kv, iv, e, s, m)
    ko_ref[...] = kv
    io_ref[...] = iv


def _cross_body(s, m, k_ref, i_ref, ko_ref, io_ref):
    desc = ((pl.program_id(0) * (2 * s)) & m) != 0
    half = s // 128
    klo, khi = k_ref[:half], k_ref[half:]
    ilo, ihi = i_ref[:half], i_ref[half:]
    swap = _gt(klo, ilo, khi, ihi) ^ desc
    ko_ref[:half] = jnp.where(swap, khi, klo)
    ko_ref[half:] = jnp.where(swap, klo, khi)
    io_ref[:half] = jnp.where(swap, ihi, ilo)
    io_ref[half:] = jnp.where(swap, ilo, ihi)


def _sort_call(body, grid, rows, K, I):
    bs = pl.BlockSpec((rows, 128), lambda i: (i, 0))
    return pl.pallas_call(
        body,
        grid=(grid,),
        in_specs=[bs, bs],
        out_specs=[bs, bs],
        out_shape=[jax.ShapeDtypeStruct((K.shape[0], 128), jnp.int32)] * 2,
        input_output_aliases={0: 0, 1: 1},
    )(K, I)


def _cross_lo_body(s, k_ref, i_ref, ko_ref, io_ref):
    half = s // 128
    klo, khi = k_ref[:half], k_ref[half:]
    ilo, ihi = i_ref[:half], i_ref[half:]
    swap = _gt(klo, ilo, khi, ihi)
    ko_ref[...] = jnp.where(swap, khi, klo)
    io_ref[...] = jnp.where(swap, ihi, ilo)


def _cross_lo(s, K, I):
    rows_in = K.shape[0]
    half = s // 128
    bs_in = pl.BlockSpec((rows_in, 128), lambda i: (i, 0))
    bs_out = pl.BlockSpec((half, 128), lambda i: (i, 0))
    return pl.pallas_call(
        functools.partial(_cross_lo_body, s),
        grid=(1,),
        in_specs=[bs_in, bs_in],
        out_specs=[bs_out, bs_out],
        out_shape=[jax.ShapeDtypeStruct((half, 128), jnp.int32)] * 2,
    )(K, I)


def _stages_down(m, s_hi):
    s, out = s_hi, []
    while s >= 1:
        out.append((s, m))
        s //= 2
    return out


def _fused_local(stages, rows, K, I):
    return _sort_call(functools.partial(_local_body, stages, rows),
                      _ROWS // rows, rows, K, I)


def _sort(K, I):
    stages = []
    m = 2
    while m <= 4096:
        stages += _stages_down(m, m // 2)
        m *= 2
    K, I = _fused_local(stages, _CHR, K, I)
    stages = []
    for m in (8192, 16384, 32768, 65536):
        stages += _stages_down(m, m // 2)
    K, I = _fused_local(stages, 512, K, I)
    K, I = _fused_local(_stages_down(131072, 65536), 1024, K, I)
    K, I = _sort_call(functools.partial(_cross_body, 131072, 262144),
                      2, 2048, K, I)
    K, I = _fused_local(_stages_down(262144, 65536), 1024, K, I)
    K, I = _cross_lo(262144, K, I)
    K, I = _cross_lo(131072, K, I)
    bs = pl.BlockSpec((1024, 128), lambda i: (i, 0))
    K, I = pl.pallas_call(
        functools.partial(_local_body, _stages_down(524288, 65536), 1024),
        grid=(1,),
        in_specs=[bs, bs],
        out_specs=[bs, bs],
        out_shape=[jax.ShapeDtypeStruct((1024, 128), jnp.int32)] * 2,
        input_output_aliases={0: 0, 1: 1},
    )(K, I)
    return K, I


def kernel(embeddings, positive_pairs, candidate_negatives):
    del positive_pairs
    cn = candidate_negatives
    pad_idx = (jnp.arange(_CP - _C, dtype=jnp.int32) * 7919) % _N
    i1 = jnp.concatenate([cn[:, 0], pad_idx])
    i2 = jnp.concatenate([cn[:, 1], pad_idx])
    e1 = _sc_gather(embeddings, i1, 600)
    e2 = _sc_gather(embeddings, i2, 600)
    K0 = _simkey(e1, e2)
    K = jnp.concatenate(
        [K0, jnp.full((_N2 - _CP,), 0x7FFFFFFF, jnp.int32)]).reshape(_ROWS, 128)
    I = jnp.arange(_N2, dtype=jnp.int32).reshape(_ROWS, 128)
    _, Is = _sort(K, I)
    top = Is.reshape(-1)[:_K]
    topp = jnp.concatenate(
        [top, (jnp.arange(_GP - _K, dtype=jnp.int32) * 13) % _C])
    g0 = _sc_gather(cn[:, 0] + 0, topp, _GP // 32)
    g1 = _sc_gather(cn[:, 1] + 0, topp, _GP // 32)
    return jnp.stack([g0[:_K], g1[:_K]], axis=1)


# speedup vs baseline: 2.3762x; 1.3069x over previous
import functools

import jax
import jax.numpy as jnp
from jax import lax
from jax.experimental import pallas as pl
from jax.experimental.pallas import tpu as pltpu
from jax.experimental.pallas import tpu_sc as plsc

_N = 100000
_D = 128
_C = 300000
_K = 90000
_CP = 307200
_N2 = 524288
_ROWS = _N2 // 128
_CHR = 32
_SBLK = 2048
_GP = 90112


def _sc_gather(table, idx, chunk):
    B = idx.shape[0]
    NC, NS = 2, 16
    NW = NC * NS
    bpw = B // NW
    nch = bpw // chunk
    dt = table.dtype
    two_d = table.ndim == 2
    row_shape = (chunk, table.shape[1]) if two_d else (chunk,)
    out_shape = (B, table.shape[1]) if two_d else (B,)
    mesh = plsc.VectorSubcoreMesh(core_axis_name="c", subcore_axis_name="s")

    @functools.partial(
        pl.kernel,
        mesh=mesh,
        out_type=jax.ShapeDtypeStruct(out_shape, dt),
        scratch_types=[
            pltpu.VMEM((chunk,), jnp.int32),
            pltpu.VMEM(row_shape, dt),
            pltpu.SemaphoreType.DMA,
        ],
    )
    def k(table_hbm, idx_hbm, out_hbm, idx_v, rows_v, sem):
        wid = lax.axis_index("s") * NC + lax.axis_index("c")
        base = wid * bpw

        def step(ci, carry):
            off = base + ci * chunk
            pltpu.sync_copy(idx_hbm.at[pl.ds(off, chunk)], idx_v)
            pltpu.async_copy(table_hbm.at[idx_v], rows_v, sem).wait()
            pltpu.sync_copy(rows_v, out_hbm.at[pl.ds(off, chunk)])
            return carry

        lax.fori_loop(0, nch, step, 0)

    return k(table, idx)


def _simkey_body(a_ref, b_ref, out_ref):
    a, b = a_ref[...], b_ref[...]
    num = jnp.sum((a * b).T, axis=0)
    na = jnp.sqrt(jnp.sum((a * a).T, axis=0))
    nb = jnp.sqrt(jnp.sum((b * b).T, axis=0))
    sim = num / jnp.maximum(na * nb, 1e-8)
    s = lax.bitcast_convert_type(sim, jnp.int32)
    t = jnp.where(s < 0, s ^ jnp.int32(0x7FFFFFFF), s)
    key = (t ^ jnp.int32(-1)).reshape(1, 1, _SBLK)
    e = (pl.program_id(0) * _SBLK
         + lax.broadcasted_iota(jnp.int32, (1, 1, _SBLK), 2))
    out_ref[...] = jnp.where(e < _C, key, jnp.int32(0x7FFFFFFF))


def _simkey(e1, e2):
    grid = _CP // _SBLK
    out = pl.pallas_call(
        _simkey_body,
        grid=(grid,),
        in_specs=[
            pl.BlockSpec((_SBLK, _D), lambda i: (i, 0)),
            pl.BlockSpec((_SBLK, _D), lambda i: (i, 0)),
        ],
        out_specs=pl.BlockSpec((1, 1, _SBLK), lambda i: (i, 0, 0)),
        out_shape=jax.ShapeDtypeStruct((grid, 1, _SBLK), jnp.int32),
    )(e1, e2)
    return out.reshape(-1)


def _gt(ka, ia, kb, ib):
    return (ka > kb) | ((ka == kb) & (ia > ib))


def _substage(kv, iv, e, s, m):
    if s >= 128:
        sr = s // 128
        kd, ku = jnp.roll(kv, -sr, axis=0), jnp.roll(kv, sr, axis=0)
        idn, iup = jnp.roll(iv, -sr, axis=0), jnp.roll(iv, sr, axis=0)
    else:
        kd, ku = jnp.roll(kv, -s, axis=1), jnp.roll(kv, s, axis=1)
        idn, iup = jnp.roll(iv, -s, axis=1), jnp.roll(iv, s, axis=1)
    lo = (e & s) == 0
    ko = jnp.where(lo, kd, ku)
    io = jnp.where(lo, idn, iup)
    desc = (e & m) != 0
    take = ((lo & _gt(kv, iv, ko, io)) | (~lo & _gt(ko, io, kv, iv))) ^ desc
    return jnp.where(take, ko, kv), jnp.where(take, io, iv)


def _local_body(stages, rows, nreal, ks_ref, is_ref, ko_ref, io_ref):
    @pl.when(pl.program_id(0) < nreal)
    def _run():
        base = pl.program_id(0) * (rows * 128)
        e = (base
             + 128 * lax.broadcasted_iota(jnp.int32, (rows, 128), 0)
             + lax.broadcasted_iota(jnp.int32, (rows, 128), 1))
        kv, iv = ks_ref[...], is_ref[...]
        for (s, m) in stages:
            kv, iv = _substage(kv, iv, e, s, m)
        ko_ref[...] = kv
        io_ref[...] = iv

    @pl.when(pl.program_id(0) >= nreal)
    def _copy():
        ko_ref[...] = ks_ref[...]
        io_ref[...] = is_ref[...]


def _cross_body(s, m, k_ref, i_ref, ko_ref, io_ref):
    desc = ((pl.program_id(0) * (2 * s)) & m) != 0
    half = s // 128
    klo, khi = k_ref[:half], k_ref[half:]
    ilo, ihi = i_ref[:half], i_ref[half:]
    swap = _gt(klo, ilo, khi, ihi) ^ desc
    ko_ref[:half] = jnp.where(swap, khi, klo)
    ko_ref[half:] = jnp.where(swap, klo, khi)
    io_ref[:half] = jnp.where(swap, ihi, ilo)
    io_ref[half:] = jnp.where(swap, ilo, ihi)


def _sort_call(body, grid, rows, K, I):
    bs = pl.BlockSpec((rows, 128), lambda i: (i, 0))
    return pl.pallas_call(
        body,
        grid=(grid,),
        in_specs=[bs, bs],
        out_specs=[bs, bs],
        out_shape=[jax.ShapeDtypeStruct((K.shape[0], 128), jnp.int32)] * 2,
        input_output_aliases={0: 0, 1: 1},
    )(K, I)


def _cross_lo_body(s, k_ref, i_ref, ko_ref, io_ref):
    half = s // 128
    klo, khi = k_ref[:half], k_ref[half:]
    ilo, ihi = i_ref[:half], i_ref[half:]
    swap = _gt(klo, ilo, khi, ihi)
    ko_ref[...] = jnp.where(swap, khi, klo)
    io_ref[...] = jnp.where(swap, ihi, ilo)


def _cross_lo(s, K, I):
    rows_in = K.shape[0]
    half = s // 128
    bs_in = pl.BlockSpec((rows_in, 128), lambda i: (i, 0))
    bs_out = pl.BlockSpec((half, 128), lambda i: (i, 0))
    return pl.pallas_call(
        functools.partial(_cross_lo_body, s),
        grid=(1,),
        in_specs=[bs_in, bs_in],
        out_specs=[bs_out, bs_out],
        out_shape=[jax.ShapeDtypeStruct((half, 128), jnp.int32)] * 2,
    )(K, I)


def _stages_down(m, s_hi):
    s, out = s_hi, []
    while s >= 1:
        out.append((s, m))
        s //= 2
    return out


def _fused_local(stages, rows, K, I, nreal=None):
    grid = _ROWS // rows
    if nreal is None:
        nreal = grid
    return _sort_call(functools.partial(_local_body, stages, rows, nreal),
                      grid, rows, K, I)


def _sort(K, I):
    stages = []
    m = 2
    while m <= 4096:
        stages += _stages_down(m, m // 2)
        m *= 2
    K, I = _fused_local(stages, _CHR, K, I, nreal=_CP // 4096)
    stages = []
    for m in (8192, 16384, 32768, 65536):
        stages += _stages_down(m, m // 2)
    K, I = _fused_local(stages, 512, K, I, nreal=5)
    K, I = _fused_local(_stages_down(131072, 65536), 1024, K, I, nreal=3)
    K, I = _sort_call(functools.partial(_cross_body, 131072, 262144),
                      2, 2048, K, I)
    K, I = _fused_local(_stages_down(262144, 65536), 1024, K, I)
    K, I = _cross_lo(262144, K, I)
    K, I = _cross_lo(131072, K, I)
    bs = pl.BlockSpec((1024, 128), lambda i: (i, 0))
    K, I = pl.pallas_call(
        functools.partial(_local_body, _stages_down(524288, 65536), 1024, 1),
        grid=(1,),
        in_specs=[bs, bs],
        out_specs=[bs, bs],
        out_shape=[jax.ShapeDtypeStruct((1024, 128), jnp.int32)] * 2,
        input_output_aliases={0: 0, 1: 1},
    )(K, I)
    return K, I


def kernel(embeddings, positive_pairs, candidate_negatives):
    del positive_pairs
    cn = candidate_negatives
    pad_idx = (jnp.arange(_CP - _C, dtype=jnp.int32) * 7919) % _N
    i1 = jnp.concatenate([cn[:, 0], pad_idx])
    i2 = jnp.concatenate([cn[:, 1], pad_idx])
    e1 = _sc_gather(embeddings, i1, 600)
    e2 = _sc_gather(embeddings, i2, 600)
    K0 = _simkey(e1, e2)
    K = jnp.concatenate(
        [K0, jnp.full((_N2 - _CP,), 0x7FFFFFFF, jnp.int32)]).reshape(_ROWS, 128)
    I = jnp.concatenate(
        [jnp.arange(_CP, dtype=jnp.int32),
         jnp.full((_N2 - _CP,), 0x7FFFFF00, jnp.int32)]).reshape(_ROWS, 128)
    _, Is = _sort(K, I)
    top = Is.reshape(-1)[:_K]
    topp = jnp.concatenate(
        [top, (jnp.arange(_GP - _K, dtype=jnp.int32) * 13) % _C])
    g0 = _sc_gather(cn[:, 0] + 0, topp, _GP // 32)
    g1 = _sc_gather(cn[:, 1] + 0, topp, _GP // 32)
    return jnp.stack([g0[:_K], g1[:_K]], axis=1)


# speedup vs baseline: 2.4531x; 1.0324x over previous
import functools

import jax
import jax.numpy as jnp
from jax import lax
from jax.experimental import pallas as pl
from jax.experimental.pallas import tpu as pltpu
from jax.experimental.pallas import tpu_sc as plsc

_N = 100000
_D = 128
_C = 300000
_K = 90000
_CP = 307200
_N2 = 524288
_ROWS = _N2 // 128
_CHR = 32
_SBLK = 2048
_GP = 90112


def _sc_gather(table, idx, chunk):
    B = idx.shape[0]
    NC, NS = 2, 16
    NW = NC * NS
    bpw = B // NW
    nch = bpw // chunk
    dt = table.dtype
    two_d = table.ndim == 2
    row_shape = (chunk, table.shape[1]) if two_d else (chunk,)
    out_shape = (B, table.shape[1]) if two_d else (B,)
    mesh = plsc.VectorSubcoreMesh(core_axis_name="c", subcore_axis_name="s")

    @functools.partial(
        pl.kernel,
        mesh=mesh,
        out_type=jax.ShapeDtypeStruct(out_shape, dt),
        scratch_types=[
            pltpu.VMEM((chunk,), jnp.int32),
            pltpu.VMEM(row_shape, dt),
            pltpu.SemaphoreType.DMA,
        ],
    )
    def k(table_hbm, idx_hbm, out_hbm, idx_v, rows_v, sem):
        wid = lax.axis_index("s") * NC + lax.axis_index("c")
        base = wid * bpw

        def step(ci, carry):
            off = base + ci * chunk
            pltpu.sync_copy(idx_hbm.at[pl.ds(off, chunk)], idx_v)
            pltpu.async_copy(table_hbm.at[idx_v], rows_v, sem).wait()
            pltpu.sync_copy(rows_v, out_hbm.at[pl.ds(off, chunk)])
            return carry

        lax.fori_loop(0, nch, step, 0)

    return k(table, idx)


def _simkey_body(a_ref, b_ref, out_ref):
    a, b = a_ref[...], b_ref[...]
    num = jnp.sum((a * b).T, axis=0)
    na = jnp.sqrt(jnp.sum((a * a).T, axis=0))
    nb = jnp.sqrt(jnp.sum((b * b).T, axis=0))
    sim = num / jnp.maximum(na * nb, 1e-8)
    s = lax.bitcast_convert_type(sim, jnp.int32)
    t = jnp.where(s < 0, s ^ jnp.int32(0x7FFFFFFF), s)
    key = (t ^ jnp.int32(-1)).reshape(1, 1, _SBLK)
    e = (pl.program_id(0) * _SBLK
         + lax.broadcasted_iota(jnp.int32, (1, 1, _SBLK), 2))
    out_ref[...] = jnp.where(e < _C, key, jnp.int32(0x7FFFFFFF))


def _simkey(e1, e2):
    grid = _CP // _SBLK
    out = pl.pallas_call(
        _simkey_body,
        grid=(grid,),
        in_specs=[
            pl.BlockSpec((_SBLK, _D), lambda i: (i, 0)),
            pl.BlockSpec((_SBLK, _D), lambda i: (i, 0)),
        ],
        out_specs=pl.BlockSpec((1, 1, _SBLK), lambda i: (i, 0, 0)),
        out_shape=jax.ShapeDtypeStruct((grid, 1, _SBLK), jnp.int32),
    )(e1, e2)
    return out.reshape(-1)


def _gt(ka, ia, kb, ib):
    return (ka > kb) | ((ka == kb) & (ia > ib))


def _substage(kv, iv, e, base, rows, s, m):
    if s >= 128:
        sr = s // 128
        g = rows // (2 * sr)
        sh = (g, 2, sr, 128)
        k4, i4 = kv.reshape(sh), iv.reshape(sh)
        klo, khi = k4[:, 0], k4[:, 1]
        ilo, ihi = i4[:, 0], i4[:, 1]
        gi = lax.broadcasted_iota(jnp.int32, (g, sr, 128), 0)
        desc = ((base + 2 * s * gi) & m) != 0
        swap = _gt(klo, ilo, khi, ihi) ^ desc
        kn = jnp.stack([jnp.where(swap, khi, klo),
                        jnp.where(swap, klo, khi)], axis=1)
        inn = jnp.stack([jnp.where(swap, ihi, ilo),
                         jnp.where(swap, ilo, ihi)], axis=1)
        return kn.reshape(rows, 128), inn.reshape(rows, 128)
    kd, ku = jnp.roll(kv, -s, axis=1), jnp.roll(kv, s, axis=1)
    idn, iup = jnp.roll(iv, -s, axis=1), jnp.roll(iv, s, axis=1)
    lo = (e & s) == 0
    ko = jnp.where(lo, kd, ku)
    io = jnp.where(lo, idn, iup)
    desc = (e & m) != 0
    take = ((lo & _gt(kv, iv, ko, io)) | (~lo & _gt(ko, io, kv, iv))) ^ desc
    return jnp.where(take, ko, kv), jnp.where(take, io, iv)


def _local_body(stages, rows, nreal, ks_ref, is_ref, ko_ref, io_ref):
    @pl.when(pl.program_id(0) < nreal)
    def _run():
        base = pl.program_id(0) * (rows * 128)
        e = (base
             + 128 * lax.broadcasted_iota(jnp.int32, (rows, 128), 0)
             + lax.broadcasted_iota(jnp.int32, (rows, 128), 1))
        kv, iv = ks_ref[...], is_ref[...]
        for (s, m) in stages:
            kv, iv = _substage(kv, iv, e, base, rows, s, m)
        ko_ref[...] = kv
        io_ref[...] = iv

    @pl.when(pl.program_id(0) >= nreal)
    def _copy():
        ko_ref[...] = ks_ref[...]
        io_ref[...] = is_ref[...]


def _cross_body(s, m, k_ref, i_ref, ko_ref, io_ref):
    desc = ((pl.program_id(0) * (2 * s)) & m) != 0
    half = s // 128
    klo, khi = k_ref[:half], k_ref[half:]
    ilo, ihi = i_ref[:half], i_ref[half:]
    swap = _gt(klo, ilo, khi, ihi) ^ desc
    ko_ref[:half] = jnp.where(swap, khi, klo)
    ko_ref[half:] = jnp.where(swap, klo, khi)
    io_ref[:half] = jnp.where(swap, ihi, ilo)
    io_ref[half:] = jnp.where(swap, ilo, ihi)


def _sort_call(body, grid, rows, K, I):
    bs = pl.BlockSpec((rows, 128), lambda i: (i, 0))
    return pl.pallas_call(
        body,
        grid=(grid,),
        in_specs=[bs, bs],
        out_specs=[bs, bs],
        out_shape=[jax.ShapeDtypeStruct((K.shape[0], 128), jnp.int32)] * 2,
        input_output_aliases={0: 0, 1: 1},
    )(K, I)


def _cross_lo_body(s, k_ref, i_ref, ko_ref, io_ref):
    half = s // 128
    klo, khi = k_ref[:half], k_ref[half:]
    ilo, ihi = i_ref[:half], i_ref[half:]
    swap = _gt(klo, ilo, khi, ihi)
    ko_ref[...] = jnp.where(swap, khi, klo)
    io_ref[...] = jnp.where(swap, ihi, ilo)


def _cross_lo(s, K, I):
    rows_in = K.shape[0]
    half = s // 128
    bs_in = pl.BlockSpec((rows_in, 128), lambda i: (i, 0))
    bs_out = pl.BlockSpec((half, 128), lambda i: (i, 0))
    return pl.pallas_call(
        functools.partial(_cross_lo_body, s),
        grid=(1,),
        in_specs=[bs_in, bs_in],
        out_specs=[bs_out, bs_out],
        out_shape=[jax.ShapeDtypeStruct((half, 128), jnp.int32)] * 2,
    )(K, I)


def _stages_down(m, s_hi):
    s, out = s_hi, []
    while s >= 1:
        out.append((s, m))
        s //= 2
    return out


def _fused_local(stages, rows, K, I, nreal=None):
    grid = _ROWS // rows
    if nreal is None:
        nreal = grid
    return _sort_call(functools.partial(_local_body, stages, rows, nreal),
                      grid, rows, K, I)


def _sort(K, I):
    stages = []
    m = 2
    while m <= 4096:
        stages += _stages_down(m, m // 2)
        m *= 2
    K, I = _fused_local(stages, _CHR, K, I, nreal=_CP // 4096)
    stages = []
    for m in (8192, 16384, 32768, 65536):
        stages += _stages_down(m, m // 2)
    K, I = _fused_local(stages, 512, K, I, nreal=5)
    K, I = _fused_local(_stages_down(131072, 65536), 1024, K, I, nreal=3)
    K, I = _sort_call(functools.partial(_cross_body, 131072, 262144),
                      2, 2048, K, I)
    K, I = _fused_local(_stages_down(262144, 65536), 1024, K, I)
    K, I = _cross_lo(262144, K, I)
    K, I = _cross_lo(131072, K, I)
    bs = pl.BlockSpec((1024, 128), lambda i: (i, 0))
    K, I = pl.pallas_call(
        functools.partial(_local_body, _stages_down(524288, 65536), 1024, 1),
        grid=(1,),
        in_specs=[bs, bs],
        out_specs=[bs, bs],
        out_shape=[jax.ShapeDtypeStruct((1024, 128), jnp.int32)] * 2,
        input_output_aliases={0: 0, 1: 1},
    )(K, I)
    return K, I


def kernel(embeddings, positive_pairs, candidate_negatives):
    del positive_pairs
    cn = candidate_negatives
    pad_idx = (jnp.arange(_CP - _C, dtype=jnp.int32) * 7919) % _N
    i1 = jnp.concatenate([cn[:, 0], pad_idx])
    i2 = jnp.concatenate([cn[:, 1], pad_idx])
    e1 = _sc_gather(embeddings, i1, 600)
    e2 = _sc_gather(embeddings, i2, 600)
    K0 = _simkey(e1, e2)
    K = jnp.concatenate(
        [K0, jnp.full((_N2 - _CP,), 0x7FFFFFFF, jnp.int32)]).reshape(_ROWS, 128)
    I = jnp.concatenate(
        [jnp.arange(_CP, dtype=jnp.int32),
         jnp.full((_N2 - _CP,), 0x7FFFFF00, jnp.int32)]).reshape(_ROWS, 128)
    _, Is = _sort(K, I)
    top = Is.reshape(-1)[:_K]
    topp = jnp.concatenate(
        [top, (jnp.arange(_GP - _K, dtype=jnp.int32) * 13) % _C])
    g0 = _sc_gather(cn[:, 0] + 0, topp, _GP // 32)
    g1 = _sc_gather(cn[:, 1] + 0, topp, _GP // 32)
    return jnp.stack([g0[:_K], g1[:_K]], axis=1)
